# baseline (speedup 1.0000x reference)
import functools

import jax
import jax.numpy as jnp
from jax import lax
from jax.experimental import pallas as pl
from jax.experimental.pallas import tpu as pltpu
from jax.experimental.pallas import tpu_sc as plsc

_B = 16384
_D = 64
_DP = 2 * _D
_NC = 2
_NS = 16
_NW = _NC * _NS
_BPW = _B // _NW
_CH = 128
_NCHUNK = _BPW // _CH

_TC_BLK = 2048


def _make_sc_gather():
    mesh = plsc.VectorSubcoreMesh(core_axis_name="c", subcore_axis_name="s")

    @functools.partial(
        pl.kernel,
        mesh=mesh,
        out_type=[
            jax.ShapeDtypeStruct((_B, _DP), jnp.float32),
            jax.ShapeDtypeStruct((_B, _DP), jnp.float32),
        ],
        scratch_types=[
            pltpu.VMEM((_NCHUNK, _CH), jnp.int32),
            pltpu.VMEM((_NCHUNK, _CH), jnp.int32),
            pltpu.VMEM((_BPW, _DP), jnp.float32),
            pltpu.SemaphoreType.DMA,
        ],
    )
    def gather_k(pidx_hbm, tidx_hbm, pemb_hbm, temb_hbm, pout_hbm, tout_hbm,
                 pidx_v, tidx_v, row_v, sem):
        wid = lax.axis_index("s") * _NC + lax.axis_index("c")
        base = wid * _BPW
        pltpu.sync_copy(pidx_hbm.at[wid], pidx_v)
        pltpu.sync_copy(tidx_hbm.at[wid], tidx_v)
        for j in range(_NCHUNK):
            pltpu.async_copy(
                pemb_hbm.at[pidx_v.at[j]], row_v.at[pl.ds(j * _CH, _CH)], sem)
        pltpu.make_async_copy(
            pemb_hbm.at[pl.ds(0, _BPW)], row_v, sem).wait()
        pltpu.sync_copy(row_v, pout_hbm.at[pl.ds(base, _BPW)])
        for j in range(_NCHUNK):
            pltpu.async_copy(
                temb_hbm.at[tidx_v.at[j]], row_v.at[pl.ds(j * _CH, _CH)], sem)
        pltpu.make_async_copy(
            temb_hbm.at[pl.ds(0, _BPW)], row_v, sem).wait()
        pltpu.sync_copy(row_v, tout_hbm.at[pl.ds(base, _BPW)])

    return gather_k


_sc_gather = _make_sc_gather()


def _tc_body(pg_ref, tg_ref, ppar_ref, tpar_ref,
             wp_ref, bp_ref, wt_ref, bt_ref, o_ref):
    ppar = ppar_ref[...]
    tpar = tpar_ref[...]
    pg = pg_ref[...]
    tg = tg_ref[...]
    p = pg[:, :_D] + (pg[:, _D:] - pg[:, :_D]) * ppar
    t = tg[:, :_D] + (tg[:, _D:] - tg[:, :_D]) * tpar
    dn = (((1,), (1,)), ((), ()))
    ph = jnp.maximum(
        lax.dot_general(p, wp_ref[...], dn,
                        preferred_element_type=jnp.float32) + bp_ref[...], 0.0)
    th = jnp.maximum(
        lax.dot_general(t, wt_ref[...], dn,
                        preferred_element_type=jnp.float32) + bt_ref[...], 0.0)
    o_ref[...] = jnp.sum(ph * th, axis=1, keepdims=True)


def _tc_score(p_g, t_g, p_par, t_par, W_p, b_p, W_t, b_t):
    grid = (_B // _TC_BLK,)
    return pl.pallas_call(
        _tc_body,
        grid=grid,
        in_specs=[
            pl.BlockSpec((_TC_BLK, _DP), lambda i: (i, 0)),
            pl.BlockSpec((_TC_BLK, _DP), lambda i: (i, 0)),
            pl.BlockSpec((_TC_BLK, 1), lambda i: (i, 0)),
            pl.BlockSpec((_TC_BLK, 1), lambda i: (i, 0)),
            pl.BlockSpec((_D, _D), lambda i: (0, 0)),
            pl.BlockSpec((1, _D), lambda i: (0, 0)),
            pl.BlockSpec((_D, _D), lambda i: (0, 0)),
            pl.BlockSpec((1, _D), lambda i: (0, 0)),
        ],
        out_specs=pl.BlockSpec((_TC_BLK, 1), lambda i: (i, 0)),
        out_shape=jax.ShapeDtypeStruct((_B, 1), jnp.float32),
    )(p_g, t_g, p_par, t_par, W_p, b_p.reshape(1, _D), W_t, b_t.reshape(1, _D))


def kernel(p_idx, t_idx, play_emb, track_emb, W_p, b_p, W_t, b_t):
    p_idx = p_idx.astype(jnp.int32)
    t_idx = t_idx.astype(jnp.int32)
    pemb2 = play_emb.reshape(play_emb.shape[0] // 2, _DP)
    temb2 = track_emb.reshape(track_emb.shape[0] // 2, _DP)
    pidx3 = (p_idx // 2).reshape(_NW, _NCHUNK, _CH)
    tidx3 = (t_idx // 2).reshape(_NW, _NCHUNK, _CH)
    p_par = (p_idx % 2).astype(jnp.float32).reshape(_B, 1)
    t_par = (t_idx % 2).astype(jnp.float32).reshape(_B, 1)
    p_g, t_g = _sc_gather(pidx3, tidx3, pemb2, temb2)
    out = _tc_score(p_g, t_g, p_par, t_par, W_p, b_p, W_t, b_t)
    return out[:, 0]


# speedup vs baseline: 1.2120x; 1.2120x over previous
import functools

import jax
import jax.numpy as jnp
from jax import lax
from jax.experimental import pallas as pl
from jax.experimental.pallas import tpu as pltpu

_B = 16384
_D = 64
_BLK = 256
_GRID = _B // _BLK


def _body(pidx_s, tidx_s, pemb_hbm, temb_hbm, wp_ref, bp_ref, wt_ref, bt_ref,
          o_ref, prow_v, trow_v, sem_p, sem_t):
    g = pl.program_id(0)
    base = g * _BLK

    def fire(r, _):
        i = pidx_s[base + r]
        j = tidx_s[base + r]
        pltpu.make_async_copy(pemb_hbm.at[pl.ds(i, 1)],
                              prow_v.at[pl.ds(r, 1)], sem_p).start()
        pltpu.make_async_copy(temb_hbm.at[pl.ds(j, 1)],
                              trow_v.at[pl.ds(r, 1)], sem_t).start()
        return 0

    lax.fori_loop(0, _BLK, fire, 0, unroll=8)
    pltpu.make_async_copy(pemb_hbm.at[pl.ds(0, _BLK)], prow_v, sem_p).wait()
    pltpu.make_async_copy(temb_hbm.at[pl.ds(0, _BLK)], trow_v, sem_t).wait()

    dn = (((1,), (1,)), ((), ()))
    ph = jnp.maximum(
        lax.dot_general(prow_v[...], wp_ref[...], dn,
                        preferred_element_type=jnp.float32) + bp_ref[...], 0.0)
    th = jnp.maximum(
        lax.dot_general(trow_v[...], wt_ref[...], dn,
                        preferred_element_type=jnp.float32) + bt_ref[...], 0.0)
    o_ref[...] = jnp.sum(ph * th, axis=1, keepdims=True)


def kernel(p_idx, t_idx, play_emb, track_emb, W_p, b_p, W_t, b_t):
    grid_spec = pltpu.PrefetchScalarGridSpec(
        num_scalar_prefetch=2,
        grid=(_GRID,),
        in_specs=[
            pl.BlockSpec(memory_space=pl.ANY),
            pl.BlockSpec(memory_space=pl.ANY),
            pl.BlockSpec((_D, _D), lambda i, *_: (0, 0)),
            pl.BlockSpec((1, _D), lambda i, *_: (0, 0)),
            pl.BlockSpec((_D, _D), lambda i, *_: (0, 0)),
            pl.BlockSpec((1, _D), lambda i, *_: (0, 0)),
        ],
        out_specs=pl.BlockSpec((_BLK, 1), lambda i, *_: (i, 0)),
        scratch_shapes=[
            pltpu.VMEM((_BLK, _D), jnp.float32),
            pltpu.VMEM((_BLK, _D), jnp.float32),
            pltpu.SemaphoreType.DMA,
            pltpu.SemaphoreType.DMA,
        ],
    )
    out = pl.pallas_call(
        _body,
        grid_spec=grid_spec,
        out_shape=jax.ShapeDtypeStruct((_B, 1), jnp.float32),
    )(p_idx.astype(jnp.int32), t_idx.astype(jnp.int32),
      play_emb, track_emb, W_p, b_p.reshape(1, _D), W_t, b_t.reshape(1, _D))
    return out[:, 0]


# speedup vs baseline: 1.2605x; 1.0400x over previous
import functools

import jax
import jax.numpy as jnp
from jax import lax
from jax.experimental import pallas as pl
from jax.experimental.pallas import tpu as pltpu
from jax.experimental.pallas import tpu_sc as plsc

_B = 16384
_D = 64
_NC = 2
_NS = 16
_NW = _NC * _NS
_BPW = _B // _NW
_CH = 128
_NCHUNK = _BPW // _CH

_GBLK = 256
_GGRID = _B // _GBLK
_TC_BLK = 2048


def _make_sc_playlist_gather():
    mesh = plsc.VectorSubcoreMesh(core_axis_name="c", subcore_axis_name="s")

    @functools.partial(
        pl.kernel,
        mesh=mesh,
        compiler_params=pltpu.CompilerParams(use_tc_tiling_on_sc=False),
        out_type=jax.ShapeDtypeStruct((_B, _D), jnp.float32),
        scratch_types=[
            pltpu.VMEM((_NCHUNK, _CH), jnp.int32),
            pltpu.VMEM((_BPW, _D), jnp.float32),
            pltpu.SemaphoreType.DMA,
        ],
    )
    def gather_k(pidx_hbm, pemb_hbm, pout_hbm, pidx_v, prow_v, sem):
        wid = lax.axis_index("s") * _NC + lax.axis_index("c")
        base = wid * _BPW
        pltpu.sync_copy(pidx_hbm.at[wid], pidx_v)
        copies = []
        for j in range(_NCHUNK):
            copies.append(pltpu.async_copy(
                pemb_hbm.at[pidx_v.at[j]], prow_v.at[pl.ds(j * _CH, _CH)], sem))
        for c in copies:
            c.wait()
        pltpu.sync_copy(prow_v, pout_hbm.at[pl.ds(base, _BPW)])

    return gather_k


_sc_playlist_gather = _make_sc_playlist_gather()


def _tc_track_gather_body(tidx_s, temb_hbm, o_ref, trow_v, sem_t):
    g = pl.program_id(0)
    base = g * _GBLK

    def fire(r, _):
        j = tidx_s[base + r]
        pltpu.make_async_copy(temb_hbm.at[pl.ds(j, 1)],
                              trow_v.at[pl.ds(r, 1)], sem_t).start()
        return 0

    lax.fori_loop(0, _GBLK, fire, 0, unroll=8)
    pltpu.make_async_copy(temb_hbm.at[pl.ds(0, _GBLK)], trow_v, sem_t).wait()
    o_ref[...] = trow_v[...]


def _tc_track_gather(t_idx, track_emb):
    grid_spec = pltpu.PrefetchScalarGridSpec(
        num_scalar_prefetch=1,
        grid=(_GGRID,),
        in_specs=[pl.BlockSpec(memory_space=pl.ANY)],
        out_specs=pl.BlockSpec((_GBLK, _D), lambda i, *_: (i, 0)),
        scratch_shapes=[
            pltpu.VMEM((_GBLK, _D), jnp.float32),
            pltpu.SemaphoreType.DMA,
        ],
    )
    return pl.pallas_call(
        _tc_track_gather_body,
        grid_spec=grid_spec,
        out_shape=jax.ShapeDtypeStruct((_B, _D), jnp.float32),
    )(t_idx, track_emb)


def _tc_mlp_body(p_ref, t_ref, wp_ref, bp_ref, wt_ref, bt_ref, o_ref):
    dn = (((1,), (1,)), ((), ()))
    ph = jnp.maximum(
        lax.dot_general(p_ref[...], wp_ref[...], dn,
                        preferred_element_type=jnp.float32) + bp_ref[...], 0.0)
    th = jnp.maximum(
        lax.dot_general(t_ref[...], wt_ref[...], dn,
                        preferred_element_type=jnp.float32) + bt_ref[...], 0.0)
    o_ref[...] = jnp.sum(ph * th, axis=1, keepdims=True)


def _tc_score(p_rows, t_rows, W_p, b_p, W_t, b_t):
    grid = (_B // _TC_BLK,)
    return pl.pallas_call(
        _tc_mlp_body,
        grid=grid,
        in_specs=[
            pl.BlockSpec((_TC_BLK, _D), lambda i: (i, 0)),
            pl.BlockSpec((_TC_BLK, _D), lambda i: (i, 0)),
            pl.BlockSpec((_D, _D), lambda i: (0, 0)),
            pl.BlockSpec((1, _D), lambda i: (0, 0)),
            pl.BlockSpec((_D, _D), lambda i: (0, 0)),
            pl.BlockSpec((1, _D), lambda i: (0, 0)),
        ],
        out_specs=pl.BlockSpec((_TC_BLK, 1), lambda i: (i, 0)),
        out_shape=jax.ShapeDtypeStruct((_B, 1), jnp.float32),
    )(p_rows, t_rows, W_p, b_p.reshape(1, _D), W_t, b_t.reshape(1, _D))


def kernel(p_idx, t_idx, play_emb, track_emb, W_p, b_p, W_t, b_t):
    pidx3 = p_idx.astype(jnp.int32).reshape(_NW, _NCHUNK, _CH)
    p_rows = _sc_playlist_gather(pidx3, play_emb)
    t_rows = _tc_track_gather(t_idx.astype(jnp.int32), track_emb)
    out = _tc_score(p_rows, t_rows, W_p, b_p, W_t, b_t)
    return out[:, 0]
